# baseline (speedup 1.0000x reference)
import functools

import jax
import jax.numpy as jnp
import numpy as np
from jax import lax
from jax.experimental import pallas as pl
from jax.experimental.pallas import tpu as pltpu
from jax.experimental.pallas import tpu_sc as plsc

N_NODES = 10000
N_EDGES = 160000
H = 8
DH = 32
HID = 256

NC = 2
NS = 16
CH = 64
NCHUNK = 158
EPT = NCHUNK * CH
E_PAD = NS * EPT
ACC_ROWS = 10048
SLAB = ACC_ROWS // NS
ROW_W = 144
BR = 400
GRID = N_NODES // BR


def _proj_block(hb, ws_ref, bs_ref, wd_ref, bd_ref, ts_ref, td_ref):
    fs = jnp.dot(hb, ws_ref[...], preferred_element_type=jnp.float32) + bs_ref[...]
    fd = jnp.dot(hb, wd_ref[...], preferred_element_type=jnp.float32) + bd_ref[...]
    z = jnp.zeros((BR, ROW_W - 128), jnp.float32)
    ts_ref[0] = fs[:, :128]
    ts_ref[1] = fs[:, 128:]
    td_ref[0] = jnp.concatenate([fd[:, :128], z], axis=1)
    td_ref[1] = jnp.concatenate([fd[:, 128:], z], axis=1)


def _embed_proj_body(gid_ref, cbo_ref, enc_ref, emb_ref, wh_ref, bh_ref,
                     ws_ref, bs_ref, wd_ref, bd_ref, h_ref, t_ref, td_ref):
    gid = gid_ref[...]
    iot = lax.broadcasted_iota(jnp.int32, (1, 32), 1)
    onehot = (gid == iot).astype(jnp.float32)
    h0a = jnp.dot(onehot, emb_ref[...], preferred_element_type=jnp.float32)
    hcat = jnp.concatenate([h0a, cbo_ref[...], enc_ref[...]], axis=1)
    y = jnp.dot(hcat, wh_ref[...], preferred_element_type=jnp.float32)
    y = y + bh_ref[...]
    hb = jnp.maximum(y, 0.01 * y)
    h_ref[...] = hb
    _proj_block(hb, ws_ref, bs_ref, wd_ref, bd_ref, t_ref, td_ref)


_W_SPECS = [
    pl.BlockSpec((256, 256), lambda i: (0, 0)),
    pl.BlockSpec((1, 256), lambda i: (0, 0)),
    pl.BlockSpec((256, 256), lambda i: (0, 0)),
    pl.BlockSpec((1, 256), lambda i: (0, 0)),
]
_HT_OUT_SPECS = [
    pl.BlockSpec((BR, 256), lambda i: (i, 0)),
    pl.BlockSpec((2, BR, 128), lambda i: (0, i, 0)),
    pl.BlockSpec((2, BR, ROW_W), lambda i: (0, i, 0)),
]
_HT_OUT_SHAPE = [
    jax.ShapeDtypeStruct((N_NODES, 256), jnp.float32),
    jax.ShapeDtypeStruct((2, N_NODES, 128), jnp.float32),
    jax.ShapeDtypeStruct((2, N_NODES, ROW_W), jnp.float32),
]

_embed_proj_call = pl.pallas_call(
    _embed_proj_body,
    grid=(GRID,),
    in_specs=[
        pl.BlockSpec((BR, 1), lambda i: (i, 0)),
        pl.BlockSpec((BR, 64), lambda i: (i, 0)),
        pl.BlockSpec((BR, 128), lambda i: (i, 0)),
        pl.BlockSpec((32, 64), lambda i: (0, 0)),
        pl.BlockSpec((256, 256), lambda i: (0, 0)),
        pl.BlockSpec((1, 256), lambda i: (0, 0)),
    ] + _W_SPECS,
    out_specs=_HT_OUT_SPECS,
    out_shape=_HT_OUT_SHAPE,
)


def _edge_body(tfs_hbm, tfd_hbm, cidx_hbm, attn_hbm, out_hbm,
               acc_sh, idx0, idx1, idx2, f0v, f1v, w0, w1,
               attn_v, is0, is1, is2, gs0, gs1, ss0, ss1):
    c = lax.axis_index("c")
    s = lax.axis_index("s")
    idxs = (idx0, idx1, idx2)
    fvs = (f0v, f1v)
    ws = (w0, w1)
    iss = (is0, is1, is2)
    gss = (gs0, gs1)
    sss = (ss0, ss1)

    pltpu.sync_copy(attn_hbm, attn_v)

    zero16 = jnp.zeros((16,), jnp.float32)

    def zrow(i, carry):
        for j in range(ROW_W // 16):
            w0[i, pl.ds(j * 16, 16)] = zero16
        return carry

    lax.fori_loop(0, CH, zrow, 0)
    zb = s * SLAB
    for r in range(SLAB // CH):
        pltpu.sync_copy(w0, acc_sh.at[pl.ds(zb + r * CH, CH), :])
    rem = SLAB % CH
    if rem:
        pltpu.sync_copy(w0.at[pl.ds(0, rem), :],
                        acc_sh.at[pl.ds(zb + (SLAB // CH) * CH, rem), :])
    plsc.subcore_barrier()

    a_vecs = []
    for h in range(4):
        row = c * 4 + h
        a_vecs.append((attn_v[row, pl.ds(0, 16)], attn_v[row, pl.ds(16, 16)]))

    lanes = lax.iota(jnp.int32, 16)
    perms = [(lanes ^ k).reshape(16, 1) for k in (8, 4, 2, 1)]
    gd = lax.GatherDimensionNumbers(
        offset_dims=(), collapsed_slice_dims=(0,), start_index_map=(0,))

    def _lane_shuffle(x, p):
        return lax.gather(x, p, gd, (1,),
                          mode=lax.GatherScatterMode.PROMISE_IN_BOUNDS)

    def load_idx_sync(k, i):
        base = s * EPT + k * CH
        pltpu.sync_copy(cidx_hbm.at[c, :, pl.ds(base, CH)], idxs[i])

    def load_idx(k, i):
        kc = jnp.minimum(k, NCHUNK - 1)
        base = s * EPT + kc * CH
        pltpu.async_copy(cidx_hbm.at[c, :, pl.ds(base, CH)], idxs[i], iss[i])

    def wait_idx(i):
        pltpu.make_async_copy(cidx_hbm.at[c, :, pl.ds(0, CH)],
                              idxs[i], iss[i]).wait()

    def start_gather(i, b):
        pltpu.async_copy(tfs_hbm.at[idxs[i].at[0]], fvs[b], gss[b])
        pltpu.async_copy(tfd_hbm.at[idxs[i].at[1]], ws[b], gss[b])

    def wait_gather(i, b):
        pltpu.make_async_copy(tfs_hbm.at[idxs[i].at[0]], fvs[b], gss[b]).wait()
        pltpu.make_async_copy(tfd_hbm.at[idxs[i].at[1]], ws[b], gss[b]).wait()

    def start_scatter(i, b):
        pltpu.async_copy(ws[b], acc_sh.at[idxs[i].at[2]], sss[b], add=True)

    def wait_scatter(i, b):
        pltpu.make_async_copy(ws[b], acc_sh.at[idxs[i].at[2]], sss[b]).wait()

    def compute(b):
        f_v = fvs[b]
        w_v = ws[b]

        def edge2(e2, ecarry):
            for u in range(4):
                e = 4 * e2 + u
                den_acc = zero16
                for h in range(4):
                    f0 = f_v[e, pl.ds(h * 32, 16)]
                    f1 = f_v[e, pl.ds(h * 32 + 16, 16)]
                    g0 = w_v[e, pl.ds(h * 32, 16)]
                    g1 = w_v[e, pl.ds(h * 32 + 16, 16)]
                    x0 = f0 + g0
                    x1 = f1 + g1
                    t0 = jnp.maximum(x0, x0 * 0.2)
                    t1 = jnp.maximum(x1, x1 * 0.2)
                    sh = t0 * a_vecs[h][0] + t1 * a_vecs[h][1]
                    for p in perms:
                        sh = sh + _lane_shuffle(sh, p)
                    ex = jnp.exp(sh)
                    w_v[e, pl.ds(h * 32, 16)] = f0 * ex
                    w_v[e, pl.ds(h * 32 + 16, 16)] = f1 * ex
                    den_acc = den_acc + jnp.where(lanes == h, ex, 0.0)
                w_v[e, pl.ds(128, 16)] = den_acc
            return ecarry

        lax.fori_loop(0, CH // 4, edge2, 0)

    load_idx_sync(0, 0)
    load_idx(1, 1)
    load_idx(2, 2)
    start_gather(0, 0)

    wait_idx(1)
    start_gather(1, 1)
    wait_gather(0, 0)
    compute(0)
    start_scatter(0, 0)

    wait_scatter(0, 0)
    load_idx(3, 0)
    wait_idx(2)
    start_gather(2, 0)
    wait_gather(1, 1)
    compute(1)
    start_scatter(1, 1)

    def six_body(kk, carry):
        for j in range(6):
            k = 2 + 6 * kk + j
            b = j % 2
            i = (2 + j) % 3
            i1 = (3 + j) % 3
            i2 = (4 + j) % 3
            nb = 1 - b
            wait_scatter(i2, nb)
            load_idx(k + 2, i2)
            wait_idx(i1)
            start_gather(i1, nb)
            wait_gather(i, b)
            compute(b)
            start_scatter(i, b)
        return carry

    lax.fori_loop(0, (NCHUNK - 2) // 6, six_body, 0)

    wait_scatter(1, 1)
    wait_gather(2, 0)
    wait_idx(0)

    plsc.subcore_barrier()

    rb = s * SLAB
    pltpu.sync_copy(acc_sh.at[pl.ds(rb, SLAB), :],
                    out_hbm.at[c, pl.ds(rb, SLAB), :])


_edge_call = pl.kernel(
    _edge_body,
    out_type=jax.ShapeDtypeStruct((NC, ACC_ROWS, ROW_W), jnp.float32),
    mesh=plsc.VectorSubcoreMesh(core_axis_name="c", subcore_axis_name="s"),
    compiler_params=pltpu.CompilerParams(use_tc_tiling_on_sc=False),
    scratch_types=[
        pltpu.VMEM_SHARED((ACC_ROWS, ROW_W), jnp.float32),
        pltpu.VMEM((3, CH), jnp.int32),
        pltpu.VMEM((3, CH), jnp.int32),
        pltpu.VMEM((3, CH), jnp.int32),
        pltpu.VMEM((CH, 128), jnp.float32),
        pltpu.VMEM((CH, 128), jnp.float32),
        pltpu.VMEM((CH, ROW_W), jnp.float32),
        pltpu.VMEM((CH, ROW_W), jnp.float32),
        pltpu.VMEM((8, 32), jnp.float32),
        pltpu.SemaphoreType.DMA,
        pltpu.SemaphoreType.DMA,
        pltpu.SemaphoreType.DMA,
        pltpu.SemaphoreType.DMA,
        pltpu.SemaphoreType.DMA,
        pltpu.SemaphoreType.DMA,
        pltpu.SemaphoreType.DMA,
    ],
)


def _post_block(do_ln, x_ref, h_ref, mavg_ref, r16_ref, g_ref, b_ref):
    x0 = x_ref[0]
    x1 = x_ref[1]
    r16 = r16_ref[...]
    den0 = jnp.maximum(
        jnp.dot(x0[:, 128:], r16, preferred_element_type=jnp.float32), 1e-9)
    den1 = jnp.maximum(
        jnp.dot(x1[:, 128:], r16, preferred_element_type=jnp.float32), 1e-9)
    h3 = jnp.concatenate([x0[:, :128] / den0, x1[:, :128] / den1], axis=1)
    h3 = h3 + h_ref[...]
    if do_ln:
        mavg = mavg_ref[...]
        mu = jnp.dot(h3, mavg, preferred_element_type=jnp.float32)
        var = jnp.dot(h3 * h3, mavg, preferred_element_type=jnp.float32) - mu * mu
        y = (h3 - mu) * lax.rsqrt(var + 1e-5) * g_ref[...] + b_ref[...]
    else:
        y = h3
    return jnp.maximum(y, 0.01 * y)


def _post_proj_body(x_ref, h_ref, mavg_ref, r16_ref, g_ref, b_ref,
                    ws_ref, bs_ref, wd_ref, bd_ref, ho_ref, t_ref, td_ref):
    hb = _post_block(True, x_ref, h_ref, mavg_ref, r16_ref, g_ref, b_ref)
    ho_ref[...] = hb
    _proj_block(hb, ws_ref, bs_ref, wd_ref, bd_ref, t_ref, td_ref)


_POST_IN_SPECS = [
    pl.BlockSpec((NC, BR, ROW_W), lambda i: (0, i, 0)),
    pl.BlockSpec((BR, 256), lambda i: (i, 0)),
    pl.BlockSpec((256, 256), lambda i: (0, 0)),
    pl.BlockSpec((16, 128), lambda i: (0, 0)),
    pl.BlockSpec((1, 256), lambda i: (0, 0)),
    pl.BlockSpec((1, 256), lambda i: (0, 0)),
]

_post_proj_call = pl.pallas_call(
    _post_proj_body,
    grid=(GRID,),
    in_specs=_POST_IN_SPECS + _W_SPECS,
    out_specs=_HT_OUT_SPECS,
    out_shape=_HT_OUT_SHAPE,
)


def _post_readout_body(x_ref, h_ref, mavg_ref, r16_ref, g_ref, b_ref,
                       inst_ref, w1, b1, w2, b2, w3, b3, w4, b4,
                       o_ref, acc_ref):
    hb = _post_block(False, x_ref, h_ref, mavg_ref, r16_ref, g_ref, b_ref)
    i = pl.program_id(0)

    @pl.when(i == 0)
    def _():
        acc_ref[...] = jnp.zeros_like(acc_ref)

    acc_ref[...] += jnp.sum(hb, axis=0, keepdims=True)

    @pl.when(i == GRID - 1)
    def _():
        hg = acc_ref[...] / float(N_NODES)
        x = jnp.concatenate([hg, inst_ref[...]], axis=1)
        x = jnp.maximum(
            jnp.dot(x, w1[...], preferred_element_type=jnp.float32) + b1[...], 0.0)
        x = jnp.maximum(
            jnp.dot(x, w2[...], preferred_element_type=jnp.float32) + b2[...], 0.0)
        x = jnp.maximum(
            jnp.dot(x, w3[...], preferred_element_type=jnp.float32) + b3[...], 0.0)
        x = jnp.dot(x, w4[...], preferred_element_type=jnp.float32) + b4[...]
        o_ref[...] = jnp.exp(x)


_post_readout_call = pl.pallas_call(
    _post_readout_body,
    grid=(GRID,),
    in_specs=_POST_IN_SPECS + [
        pl.BlockSpec((1, 32), lambda i: (0, 0)),
        pl.BlockSpec((288, 256), lambda i: (0, 0)),
        pl.BlockSpec((1, 256), lambda i: (0, 0)),
        pl.BlockSpec((256, 256), lambda i: (0, 0)),
        pl.BlockSpec((1, 256), lambda i: (0, 0)),
        pl.BlockSpec((256, 256), lambda i: (0, 0)),
        pl.BlockSpec((1, 256), lambda i: (0, 0)),
        pl.BlockSpec((256, 1), lambda i: (0, 0)),
        pl.BlockSpec((1, 1), lambda i: (0, 0)),
    ],
    out_specs=pl.BlockSpec((1, 1), lambda i: (0, 0)),
    out_shape=jax.ShapeDtypeStruct((1, 1), jnp.float32),
    scratch_shapes=[pltpu.VMEM((1, 256), jnp.float32)],
)


_MAVG_np = np.kron(np.eye(8), np.full((32, 32), 1.0 / 32.0)).astype(np.float32)
_R16_np = np.zeros((16, 128), dtype=np.float32)
for _j in range(4):
    _R16_np[_j, _j * 32:(_j + 1) * 32] = 1.0


def kernel(op_gid, cbo, enc, edge_index, inst_feat, params):
    src = edge_index[0].astype(jnp.int32)
    dst = edge_index[1].astype(jnp.int32)
    pad = E_PAD - N_EDGES
    srcp = jnp.concatenate([src, jnp.zeros((pad,), jnp.int32)])
    dstp = jnp.concatenate([dst, jnp.zeros((pad,), jnp.int32)])
    sdst = jnp.concatenate([dst, jnp.full((pad,), N_NODES, jnp.int32)])
    core_off = (jnp.arange(NC, dtype=jnp.int32) * N_NODES)[:, None]
    gsrc = srcp[None, :] + core_off
    gdst = dstp[None, :] + core_off
    cidx = jnp.stack(
        [gsrc, gdst, jnp.broadcast_to(sdst[None, :], (NC, E_PAD))], axis=1)

    mavg = jnp.asarray(_MAVG_np)
    r16 = jnp.asarray(_R16_np)
    layers = params["layers"]

    def wargs(p):
        return (p["Wsrc"], p["bsrc"].reshape(1, 256),
                p["Wdst"], p["bdst"].reshape(1, 256))

    h, t, td = _embed_proj_call(
        op_gid.reshape(N_NODES, 1).astype(jnp.int32), cbo, enc,
        params["emb"], params["W_h"], params["b_h"].reshape(1, 256),
        *wargs(layers[0]))

    for i in range(3):
        acc = _edge_call(t.reshape(2 * N_NODES, 128),
                         td.reshape(2 * N_NODES, ROW_W), cidx,
                         layers[i]["attn"])
        ln = params["ln"][i]
        g = jnp.tile(ln["g"], H).reshape(1, 256)
        b = jnp.tile(ln["b"], H).reshape(1, 256)
        h, t, td = _post_proj_call(acc, h, mavg, r16, g, b,
                                   *wargs(layers[i + 1]))

    acc = _edge_call(t.reshape(2 * N_NODES, 128),
                     td.reshape(2 * N_NODES, ROW_W), cidx, layers[3]["attn"])

    zed = jnp.zeros((1, 256), jnp.float32)
    mlp = params["mlp"]
    return _post_readout_call(
        acc, h, mavg, r16, zed, zed, inst_feat,
        mlp[0][0], mlp[0][1].reshape(1, 256),
        mlp[1][0], mlp[1][1].reshape(1, 256),
        mlp[2][0], mlp[2][1].reshape(1, 256),
        mlp[3][0], mlp[3][1].reshape(1, 1),
    )


# speedup vs baseline: 2.6032x; 2.6032x over previous
import functools

import jax
import jax.numpy as jnp
import numpy as np
from jax import lax
from jax.experimental import pallas as pl
from jax.experimental.pallas import tpu as pltpu
from jax.experimental.pallas import tpu_sc as plsc

N_NODES = 10000
N_EDGES = 160000
H = 8
DH = 32
HID = 256

NC = 2
NS = 16
CH = 64
NCHUNK = 158
EPT = NCHUNK * CH
E_PAD = NS * EPT
ACC_ROWS = 10048
SLAB = ACC_ROWS // NS
ROW_W = 144
BR = 400
GRID = N_NODES // BR


def _proj_block(hb, ws_ref, bs_ref, wd_ref, bd_ref, ts_ref, td_ref):
    fs = jnp.dot(hb, ws_ref[...], preferred_element_type=jnp.float32) + bs_ref[...]
    fd = jnp.dot(hb, wd_ref[...], preferred_element_type=jnp.float32) + bd_ref[...]
    z = jnp.zeros((BR, ROW_W - 128), jnp.float32)
    ts_ref[0] = jnp.concatenate([fs[:, :128], z], axis=1)
    ts_ref[1] = jnp.concatenate([fs[:, 128:], z], axis=1)
    td_ref[0] = jnp.concatenate([fd[:, :128], z], axis=1)
    td_ref[1] = jnp.concatenate([fd[:, 128:], z], axis=1)


def _embed_proj_body(gid_ref, cbo_ref, enc_ref, emb_ref, wh_ref, bh_ref,
                     ws_ref, bs_ref, wd_ref, bd_ref, h_ref, t_ref, td_ref):
    gid = gid_ref[...]
    iot = lax.broadcasted_iota(jnp.int32, (1, 32), 1)
    onehot = (gid == iot).astype(jnp.float32)
    h0a = jnp.dot(onehot, emb_ref[...], preferred_element_type=jnp.float32)
    hcat = jnp.concatenate([h0a, cbo_ref[...], enc_ref[...]], axis=1)
    y = jnp.dot(hcat, wh_ref[...], preferred_element_type=jnp.float32)
    y = y + bh_ref[...]
    hb = jnp.maximum(y, 0.01 * y)
    h_ref[...] = hb
    _proj_block(hb, ws_ref, bs_ref, wd_ref, bd_ref, t_ref, td_ref)


_W_SPECS = [
    pl.BlockSpec((256, 256), lambda i: (0, 0)),
    pl.BlockSpec((1, 256), lambda i: (0, 0)),
    pl.BlockSpec((256, 256), lambda i: (0, 0)),
    pl.BlockSpec((1, 256), lambda i: (0, 0)),
]
_HT_OUT_SPECS = [
    pl.BlockSpec((BR, 256), lambda i: (i, 0)),
    pl.BlockSpec((2, BR, ROW_W), lambda i: (0, i, 0)),
    pl.BlockSpec((2, BR, ROW_W), lambda i: (0, i, 0)),
]
_HT_OUT_SHAPE = [
    jax.ShapeDtypeStruct((N_NODES, 256), jnp.float32),
    jax.ShapeDtypeStruct((2, N_NODES, ROW_W), jnp.float32),
    jax.ShapeDtypeStruct((2, N_NODES, ROW_W), jnp.float32),
]

_embed_proj_call = pl.pallas_call(
    _embed_proj_body,
    grid=(GRID,),
    in_specs=[
        pl.BlockSpec((BR, 1), lambda i: (i, 0)),
        pl.BlockSpec((BR, 64), lambda i: (i, 0)),
        pl.BlockSpec((BR, 128), lambda i: (i, 0)),
        pl.BlockSpec((32, 64), lambda i: (0, 0)),
        pl.BlockSpec((256, 256), lambda i: (0, 0)),
        pl.BlockSpec((1, 256), lambda i: (0, 0)),
    ] + _W_SPECS,
    out_specs=_HT_OUT_SPECS,
    out_shape=_HT_OUT_SHAPE,
)


def _edge_body(tfs_hbm, tfd_hbm, cidx_hbm, attn_hbm, out_hbm,
               acc_sh, idx0, idx1, idx2, f0v, f1v, w0, w1,
               attn_v, is0, is1, is2, gs0, gs1, ss0, ss1):
    c = lax.axis_index("c")
    s = lax.axis_index("s")
    idxs = (idx0, idx1, idx2)
    fvs = (f0v, f1v)
    ws = (w0, w1)
    iss = (is0, is1, is2)
    gss = (gs0, gs1)
    sss = (ss0, ss1)

    pltpu.sync_copy(attn_hbm, attn_v)

    zero16 = jnp.zeros((16,), jnp.float32)

    def zrow(i, carry):
        for j in range(ROW_W // 16):
            w0[i, pl.ds(j * 16, 16)] = zero16
        return carry

    lax.fori_loop(0, CH, zrow, 0)
    zb = s * SLAB
    for r in range(SLAB // CH):
        pltpu.sync_copy(w0, acc_sh.at[pl.ds(zb + r * CH, CH), :])
    rem = SLAB % CH
    if rem:
        pltpu.sync_copy(w0.at[pl.ds(0, rem), :],
                        acc_sh.at[pl.ds(zb + (SLAB // CH) * CH, rem), :])
    plsc.subcore_barrier()

    a_vecs = []
    for h in range(4):
        row = c * 4 + h
        a_vecs.append((attn_v[row, pl.ds(0, 16)], attn_v[row, pl.ds(16, 16)]))

    lanes = lax.iota(jnp.int32, 16)
    perms = [(lanes ^ k).reshape(16, 1) for k in (8, 4, 2, 1)]
    gd = lax.GatherDimensionNumbers(
        offset_dims=(), collapsed_slice_dims=(0,), start_index_map=(0,))

    def _lane_shuffle(x, p):
        return lax.gather(x, p, gd, (1,),
                          mode=lax.GatherScatterMode.PROMISE_IN_BOUNDS)

    def load_idx_sync(k, i):
        base = s * EPT + k * CH
        pltpu.sync_copy(cidx_hbm.at[c, :, pl.ds(base, CH)], idxs[i])

    def load_idx(k, i):
        kc = jnp.minimum(k, NCHUNK - 1)
        base = s * EPT + kc * CH
        pltpu.async_copy(cidx_hbm.at[c, :, pl.ds(base, CH)], idxs[i], iss[i])

    def wait_idx(i):
        pltpu.make_async_copy(cidx_hbm.at[c, :, pl.ds(0, CH)],
                              idxs[i], iss[i]).wait()

    def start_gather(i, b):
        pltpu.async_copy(tfs_hbm.at[idxs[i].at[0]], fvs[b], gss[b])
        pltpu.async_copy(tfd_hbm.at[idxs[i].at[1]], ws[b], gss[b])

    def wait_gather(i, b):
        pltpu.make_async_copy(tfs_hbm.at[idxs[i].at[0]], fvs[b], gss[b]).wait()
        pltpu.make_async_copy(tfd_hbm.at[idxs[i].at[1]], ws[b], gss[b]).wait()

    def start_scatter(i, b):
        pltpu.async_copy(ws[b], acc_sh.at[idxs[i].at[2]], sss[b], add=True)

    def wait_scatter(i, b):
        pltpu.make_async_copy(ws[b], acc_sh.at[idxs[i].at[2]], sss[b]).wait()

    def compute(b):
        f_v = fvs[b]
        w_v = ws[b]

        def edge2(e2, ecarry):
            for u in range(4):
                e = 4 * e2 + u
                den_acc = zero16
                for h in range(4):
                    f0 = f_v[e, pl.ds(h * 32, 16)]
                    f1 = f_v[e, pl.ds(h * 32 + 16, 16)]
                    g0 = w_v[e, pl.ds(h * 32, 16)]
                    g1 = w_v[e, pl.ds(h * 32 + 16, 16)]
                    x0 = f0 + g0
                    x1 = f1 + g1
                    t0 = jnp.maximum(x0, x0 * 0.2)
                    t1 = jnp.maximum(x1, x1 * 0.2)
                    sh = t0 * a_vecs[h][0] + t1 * a_vecs[h][1]
                    for p in perms:
                        sh = sh + _lane_shuffle(sh, p)
                    ex = jnp.exp(sh)
                    w_v[e, pl.ds(h * 32, 16)] = f0 * ex
                    w_v[e, pl.ds(h * 32 + 16, 16)] = f1 * ex
                    den_acc = den_acc + jnp.where(lanes == h, ex, 0.0)
                w_v[e, pl.ds(128, 16)] = den_acc
            return ecarry

        lax.fori_loop(0, CH // 4, edge2, 0)

    load_idx_sync(0, 0)
    load_idx(1, 1)
    load_idx(2, 2)
    start_gather(0, 0)

    wait_idx(1)
    start_gather(1, 1)
    wait_gather(0, 0)
    compute(0)
    start_scatter(0, 0)

    wait_scatter(0, 0)
    load_idx(3, 0)
    wait_idx(2)
    start_gather(2, 0)
    wait_gather(1, 1)
    compute(1)
    start_scatter(1, 1)

    def six_body(kk, carry):
        for j in range(6):
            k = 2 + 6 * kk + j
            b = j % 2
            i = (2 + j) % 3
            i1 = (3 + j) % 3
            i2 = (4 + j) % 3
            nb = 1 - b
            wait_scatter(i2, nb)
            load_idx(k + 2, i2)
            wait_idx(i1)
            start_gather(i1, nb)
            wait_gather(i, b)
            compute(b)
            start_scatter(i, b)
        return carry

    lax.fori_loop(0, (NCHUNK - 2) // 6, six_body, 0)

    wait_scatter(1, 1)
    wait_gather(2, 0)
    wait_idx(0)

    plsc.subcore_barrier()

    rb = s * SLAB
    pltpu.sync_copy(acc_sh.at[pl.ds(rb, SLAB), :],
                    out_hbm.at[c, pl.ds(rb, SLAB), :])


_edge_call = pl.kernel(
    _edge_body,
    out_type=jax.ShapeDtypeStruct((NC, ACC_ROWS, ROW_W), jnp.float32),
    mesh=plsc.VectorSubcoreMesh(core_axis_name="c", subcore_axis_name="s"),
    compiler_params=pltpu.CompilerParams(use_tc_tiling_on_sc=False),
    scratch_types=[
        pltpu.VMEM_SHARED((ACC_ROWS, ROW_W), jnp.float32),
        pltpu.VMEM((3, CH), jnp.int32),
        pltpu.VMEM((3, CH), jnp.int32),
        pltpu.VMEM((3, CH), jnp.int32),
        pltpu.VMEM((CH, ROW_W), jnp.float32),
        pltpu.VMEM((CH, ROW_W), jnp.float32),
        pltpu.VMEM((CH, ROW_W), jnp.float32),
        pltpu.VMEM((CH, ROW_W), jnp.float32),
        pltpu.VMEM((8, 32), jnp.float32),
        pltpu.SemaphoreType.DMA,
        pltpu.SemaphoreType.DMA,
        pltpu.SemaphoreType.DMA,
        pltpu.SemaphoreType.DMA,
        pltpu.SemaphoreType.DMA,
        pltpu.SemaphoreType.DMA,
        pltpu.SemaphoreType.DMA,
    ],
)


def _post_block(do_ln, x_ref, h_ref, mavg_ref, r16_ref, g_ref, b_ref):
    x0 = x_ref[0]
    x1 = x_ref[1]
    r16 = r16_ref[...]
    den0 = jnp.maximum(
        jnp.dot(x0[:, 128:], r16, preferred_element_type=jnp.float32), 1e-9)
    den1 = jnp.maximum(
        jnp.dot(x1[:, 128:], r16, preferred_element_type=jnp.float32), 1e-9)
    h3 = jnp.concatenate([x0[:, :128] / den0, x1[:, :128] / den1], axis=1)
    h3 = h3 + h_ref[...]
    if do_ln:
        mavg = mavg_ref[...]
        mu = jnp.dot(h3, mavg, preferred_element_type=jnp.float32)
        var = jnp.dot(h3 * h3, mavg, preferred_element_type=jnp.float32) - mu * mu
        y = (h3 - mu) * lax.rsqrt(var + 1e-5) * g_ref[...] + b_ref[...]
    else:
        y = h3
    return jnp.maximum(y, 0.01 * y)


def _post_proj_body(x_ref, h_ref, mavg_ref, r16_ref, g_ref, b_ref,
                    ws_ref, bs_ref, wd_ref, bd_ref, ho_ref, t_ref, td_ref):
    hb = _post_block(True, x_ref, h_ref, mavg_ref, r16_ref, g_ref, b_ref)
    ho_ref[...] = hb
    _proj_block(hb, ws_ref, bs_ref, wd_ref, bd_ref, t_ref, td_ref)


_POST_IN_SPECS = [
    pl.BlockSpec((NC, BR, ROW_W), lambda i: (0, i, 0)),
    pl.BlockSpec((BR, 256), lambda i: (i, 0)),
    pl.BlockSpec((256, 256), lambda i: (0, 0)),
    pl.BlockSpec((16, 128), lambda i: (0, 0)),
    pl.BlockSpec((1, 256), lambda i: (0, 0)),
    pl.BlockSpec((1, 256), lambda i: (0, 0)),
]

_post_proj_call = pl.pallas_call(
    _post_proj_body,
    grid=(GRID,),
    in_specs=_POST_IN_SPECS + _W_SPECS,
    out_specs=_HT_OUT_SPECS,
    out_shape=_HT_OUT_SHAPE,
)


def _post_readout_body(x_ref, h_ref, mavg_ref, r16_ref, g_ref, b_ref,
                       inst_ref, w1, b1, w2, b2, w3, b3, w4, b4,
                       o_ref, acc_ref):
    hb = _post_block(False, x_ref, h_ref, mavg_ref, r16_ref, g_ref, b_ref)
    i = pl.program_id(0)

    @pl.when(i == 0)
    def _():
        acc_ref[...] = jnp.zeros_like(acc_ref)

    acc_ref[...] += jnp.sum(hb, axis=0, keepdims=True)

    @pl.when(i == GRID - 1)
    def _():
        hg = acc_ref[...] / float(N_NODES)
        x = jnp.concatenate([hg, inst_ref[...]], axis=1)
        x = jnp.maximum(
            jnp.dot(x, w1[...], preferred_element_type=jnp.float32) + b1[...], 0.0)
        x = jnp.maximum(
            jnp.dot(x, w2[...], preferred_element_type=jnp.float32) + b2[...], 0.0)
        x = jnp.maximum(
            jnp.dot(x, w3[...], preferred_element_type=jnp.float32) + b3[...], 0.0)
        x = jnp.dot(x, w4[...], preferred_element_type=jnp.float32) + b4[...]
        o_ref[...] = jnp.exp(x)


_post_readout_call = pl.pallas_call(
    _post_readout_body,
    grid=(GRID,),
    in_specs=_POST_IN_SPECS + [
        pl.BlockSpec((1, 32), lambda i: (0, 0)),
        pl.BlockSpec((288, 256), lambda i: (0, 0)),
        pl.BlockSpec((1, 256), lambda i: (0, 0)),
        pl.BlockSpec((256, 256), lambda i: (0, 0)),
        pl.BlockSpec((1, 256), lambda i: (0, 0)),
        pl.BlockSpec((256, 256), lambda i: (0, 0)),
        pl.BlockSpec((1, 256), lambda i: (0, 0)),
        pl.BlockSpec((256, 1), lambda i: (0, 0)),
        pl.BlockSpec((1, 1), lambda i: (0, 0)),
    ],
    out_specs=pl.BlockSpec((1, 1), lambda i: (0, 0)),
    out_shape=jax.ShapeDtypeStruct((1, 1), jnp.float32),
    scratch_shapes=[pltpu.VMEM((1, 256), jnp.float32)],
)


_MAVG_np = np.kron(np.eye(8), np.full((32, 32), 1.0 / 32.0)).astype(np.float32)
_R16_np = np.zeros((16, 128), dtype=np.float32)
for _j in range(4):
    _R16_np[_j, _j * 32:(_j + 1) * 32] = 1.0


def kernel(op_gid, cbo, enc, edge_index, inst_feat, params):
    src = edge_index[0].astype(jnp.int32)
    dst = edge_index[1].astype(jnp.int32)
    pad = E_PAD - N_EDGES
    srcp = jnp.concatenate([src, jnp.zeros((pad,), jnp.int32)])
    dstp = jnp.concatenate([dst, jnp.zeros((pad,), jnp.int32)])
    sdst = jnp.concatenate([dst, jnp.full((pad,), N_NODES, jnp.int32)])
    core_off = (jnp.arange(NC, dtype=jnp.int32) * N_NODES)[:, None]
    gsrc = srcp[None, :] + core_off
    gdst = dstp[None, :] + core_off
    cidx = jnp.stack(
        [gsrc, gdst, jnp.broadcast_to(sdst[None, :], (NC, E_PAD))], axis=1)

    mavg = jnp.asarray(_MAVG_np)
    r16 = jnp.asarray(_R16_np)
    layers = params["layers"]

    def wargs(p):
        return (p["Wsrc"], p["bsrc"].reshape(1, 256),
                p["Wdst"], p["bdst"].reshape(1, 256))

    h, t, td = _embed_proj_call(
        op_gid.reshape(N_NODES, 1).astype(jnp.int32), cbo, enc,
        params["emb"], params["W_h"], params["b_h"].reshape(1, 256),
        *wargs(layers[0]))

    for i in range(3):
        acc = _edge_call(t.reshape(2 * N_NODES, ROW_W),
                         td.reshape(2 * N_NODES, ROW_W), cidx,
                         layers[i]["attn"])
        ln = params["ln"][i]
        g = jnp.tile(ln["g"], H).reshape(1, 256)
        b = jnp.tile(ln["b"], H).reshape(1, 256)
        h, t, td = _post_proj_call(acc, h, mavg, r16, g, b,
                                   *wargs(layers[i + 1]))

    acc = _edge_call(t.reshape(2 * N_NODES, ROW_W),
                     td.reshape(2 * N_NODES, ROW_W), cidx, layers[3]["attn"])

    zed = jnp.zeros((1, 256), jnp.float32)
    mlp = params["mlp"]
    return _post_readout_call(
        acc, h, mavg, r16, zed, zed, inst_feat,
        mlp[0][0], mlp[0][1].reshape(1, 256),
        mlp[1][0], mlp[1][1].reshape(1, 256),
        mlp[2][0], mlp[2][1].reshape(1, 256),
        mlp[3][0], mlp[3][1].reshape(1, 1),
    )
